# baseline (speedup 1.0000x reference)
import jax
import jax.numpy as jnp
from jax.experimental import pallas as pl

DECK = 8192
EMB = 1024
HID = 2 * EMB + 2
NCARDS = 1024
NHIST = 512
NPLAYERS = 6
BLK_A = 1024
BLK_B = 512
SUCCEEDS = 100.0
GOOD_DECLARE = 150.0
I_PLAYER = 2


def _embed_body(cards_ref, hist_c_ref, hist_p_ref, score_ref, ptab_ref,
                tbl_ref, fe_ref, counts_ref):
    i = pl.program_id(0)
    col = jax.lax.broadcasted_iota(jnp.int32, (BLK_A, 1), 0) + i * BLK_A
    cards = cards_ref[...]
    own_cnt = jnp.sum((col == cards).astype(jnp.float32), axis=1,
                      keepdims=True)
    hist_c = hist_c_ref[...]
    hist_cnt = jnp.sum((col == hist_c).astype(jnp.float32), axis=1,
                       keepdims=True)
    counts_ref[...] = own_cnt
    tbl = tbl_ref[...]
    own_part = jax.lax.dot_general(
        tbl, own_cnt, (((0,), (0,)), ((), ())),
        preferred_element_type=jnp.float32)
    hist_part = jax.lax.dot_general(
        tbl, hist_cnt, (((0,), (0,)), ((), ())),
        preferred_element_type=jnp.float32)

    @pl.when(i == 0)
    def _():
        hp = hist_p_ref[...]
        pidx = jax.lax.rem(hp, NPLAYERS)
        prow = jax.lax.broadcasted_iota(jnp.int32, (NPLAYERS, 1), 0)
        pcnt = jnp.sum((prow == pidx).astype(jnp.float32), axis=1,
                       keepdims=True)
        ppart = jax.lax.dot_general(
            ptab_ref[...], pcnt, (((0,), (0,)), ((), ())),
            preferred_element_type=jnp.float32)
        fe_ref[0:EMB, :] = own_part
        fe_ref[EMB:2 * EMB, :] = hist_part + ppart
        fe_ref[2 * EMB:2 * EMB + 1, :] = score_ref[...]
        fe_ref[2 * EMB + 1:2 * EMB + 2, :] = jnp.full((1, 1), float(I_PLAYER),
                                                      jnp.float32)

    @pl.when(i > 0)
    def _():
        fe_ref[0:EMB, :] += own_part
        fe_ref[EMB:2 * EMB, :] += hist_part

    @pl.when(i == DECK // BLK_A - 1)
    def _():
        fe_ref[...] = jnp.maximum(fe_ref[...], 0.0)


def _matvec_body(fe_ref, wa_ref, ba_ref, wd_ref, bd_ref,
                 ask_ref, dec_ref, ssq_ref):
    i = pl.program_id(0)
    fe = fe_ref[...]
    a = jnp.tanh(jax.lax.dot_general(
        wa_ref[...], fe, (((1,), (0,)), ((), ())),
        preferred_element_type=jnp.float32) + ba_ref[...])
    d = jnp.tanh(jax.lax.dot_general(
        wd_ref[...], fe, (((1,), (0,)), ((), ())),
        preferred_element_type=jnp.float32) + bd_ref[...])
    ask_ref[...] = a
    dec_ref[...] = d
    vals = jnp.concatenate([jnp.sum(a * a).reshape(1, 1),
                            jnp.sum(d * d).reshape(1, 1)], axis=1)

    @pl.when(i == 0)
    def _():
        ssq_ref[...] = vals

    @pl.when(i > 0)
    def _():
        ssq_ref[...] += vals


def _final_body(ask2_ref, dec2_ref, cnt2_ref, ssq_ref, fe_ref,
                wap_ref, bap_ref, wdp_ref, bdp_ref, decl_ref,
                wsuit_ref, bsuit_ref,
                askm_ref, suit_ref, scal_ref):
    fe = fe_ref[...]
    a = jnp.tanh(jax.lax.dot_general(
        wap_ref[...], fe, (((1,), (0,)), ((), ())),
        preferred_element_type=jnp.float32) + bap_ref[...])
    q = jnp.tanh(jax.lax.dot_general(
        wdp_ref[...], fe, (((1,), (0,)), ((), ())),
        preferred_element_type=jnp.float32) + bdp_ref[...])
    ssq = ssq_ref[...]
    na = jnp.sqrt(jnp.sum(a * a))
    nq = jnp.sqrt(jnp.sum(q * q))
    nc = jnp.sqrt(ssq[0, 0])
    nd = jnp.sqrt(ssq[0, 1])
    scale_a = SUCCEEDS / (na * nc + 1e-12)
    scale_d = 1.0 / (nq * nd + 1e-12)

    c2d = ask2_ref[...]
    d2d = dec2_ref[...]
    cnt2 = cnt2_ref[...]
    inhand = cnt2 > 0.0
    sp = jnp.sum(cnt2, axis=1, keepdims=True) > 0.0
    ok = jnp.logical_and(jnp.broadcast_to(sp, (128, 64)),
                         jnp.logical_not(inhand))

    ask_score = jnp.float32(-jnp.inf)
    for r in range(3):
        row = jnp.where(ok, scale_a * a[r, 0] * c2d, -SUCCEEDS)
        askm_ref[r, :, :] = row
        ask_score = jnp.maximum(ask_score, jnp.max(row))

    suit_max = None
    for r in range(3):
        over = 1.0 if r == (I_PLAYER % 3) else -1.0
        rowv = jnp.where(inhand, over, scale_d * q[r, 0] * d2d)
        suit_max = rowv if suit_max is None else jnp.maximum(suit_max, rowv)

    ss = jnp.sum(suit_max * wsuit_ref[...], axis=1, keepdims=True)
    ss = ss + bsuit_ref[0, 0]
    nss = jnp.sqrt(jnp.sum(ss * ss))
    ss = ss / (nss + 1e-12) * GOOD_DECLARE
    decl = decl_ref[...]
    srow = jax.lax.broadcasted_iota(jnp.int32, (128, 1), 0)
    is_decl = jnp.sum((srow == decl).astype(jnp.int32), axis=1,
                      keepdims=True) > 0
    ss = jnp.where(is_decl, -GOOD_DECLARE, ss)
    suit_ref[...] = ss
    declare_score = jnp.max(ss)
    scal_ref[...] = jnp.concatenate(
        [ask_score.reshape(1, 1), declare_score.reshape(1, 1)], axis=1)


def kernel(score, history, cards, declared_suits, cards_table, players_table,
           W_ask_cards, b_ask_cards, W_ask_player, b_ask_player,
           W_dec_cards, b_dec_cards, W_dec_player, b_dec_player,
           W_suit, b_suit):
    cards2 = cards.reshape(1, NCARDS)
    hist_c = history[:, 1].reshape(1, NHIST)
    hist_p = history[:, 0].reshape(1, NHIST)
    score2 = score.reshape(1, 1)

    na = DECK // BLK_A
    fe, counts = pl.pallas_call(
        _embed_body,
        grid=(na,),
        in_specs=[
            pl.BlockSpec((1, NCARDS), lambda i: (0, 0)),
            pl.BlockSpec((1, NHIST), lambda i: (0, 0)),
            pl.BlockSpec((1, NHIST), lambda i: (0, 0)),
            pl.BlockSpec((1, 1), lambda i: (0, 0)),
            pl.BlockSpec((NPLAYERS, EMB), lambda i: (0, 0)),
            pl.BlockSpec((BLK_A, EMB), lambda i: (i, 0)),
        ],
        out_specs=[
            pl.BlockSpec((HID, 1), lambda i: (0, 0)),
            pl.BlockSpec((BLK_A, 1), lambda i: (i, 0)),
        ],
        out_shape=[
            jax.ShapeDtypeStruct((HID, 1), jnp.float32),
            jax.ShapeDtypeStruct((DECK, 1), jnp.float32),
        ],
    )(cards2, hist_c, hist_p, score2, players_table, cards_table)

    nb = DECK // BLK_B
    ask_pred, dec_pred, ssq = pl.pallas_call(
        _matvec_body,
        grid=(nb,),
        in_specs=[
            pl.BlockSpec((HID, 1), lambda i: (0, 0)),
            pl.BlockSpec((BLK_B, HID), lambda i: (i, 0)),
            pl.BlockSpec((BLK_B, 1), lambda i: (i, 0)),
            pl.BlockSpec((BLK_B, HID), lambda i: (i, 0)),
            pl.BlockSpec((BLK_B, 1), lambda i: (i, 0)),
        ],
        out_specs=[
            pl.BlockSpec((BLK_B, 1), lambda i: (i, 0)),
            pl.BlockSpec((BLK_B, 1), lambda i: (i, 0)),
            pl.BlockSpec((1, 2), lambda i: (0, 0)),
        ],
        out_shape=[
            jax.ShapeDtypeStruct((DECK, 1), jnp.float32),
            jax.ShapeDtypeStruct((DECK, 1), jnp.float32),
            jax.ShapeDtypeStruct((1, 2), jnp.float32),
        ],
    )(fe, W_ask_cards, b_ask_cards.reshape(DECK, 1),
      W_dec_cards, b_dec_cards.reshape(DECK, 1))

    askm, ss, scal = pl.pallas_call(
        _final_body,
        grid=(1,),
        in_specs=[
            pl.BlockSpec((128, 64), lambda i: (0, 0)),
            pl.BlockSpec((128, 64), lambda i: (0, 0)),
            pl.BlockSpec((128, 64), lambda i: (0, 0)),
            pl.BlockSpec((1, 2), lambda i: (0, 0)),
            pl.BlockSpec((HID, 1), lambda i: (0, 0)),
            pl.BlockSpec((3, HID), lambda i: (0, 0)),
            pl.BlockSpec((3, 1), lambda i: (0, 0)),
            pl.BlockSpec((3, HID), lambda i: (0, 0)),
            pl.BlockSpec((3, 1), lambda i: (0, 0)),
            pl.BlockSpec((1, 8), lambda i: (0, 0)),
            pl.BlockSpec((1, 64), lambda i: (0, 0)),
            pl.BlockSpec((1, 1), lambda i: (0, 0)),
        ],
        out_specs=[
            pl.BlockSpec((3, 128, 64), lambda i: (0, 0, 0)),
            pl.BlockSpec((128, 1), lambda i: (0, 0)),
            pl.BlockSpec((1, 2), lambda i: (0, 0)),
        ],
        out_shape=[
            jax.ShapeDtypeStruct((3, 128, 64), jnp.float32),
            jax.ShapeDtypeStruct((128, 1), jnp.float32),
            jax.ShapeDtypeStruct((1, 2), jnp.float32),
        ],
    )(ask_pred.reshape(128, 64), dec_pred.reshape(128, 64),
      counts.reshape(128, 64), ssq, fe,
      W_ask_player, b_ask_player.reshape(3, 1),
      W_dec_player, b_dec_player.reshape(3, 1),
      declared_suits.reshape(1, 8), W_suit, b_suit.reshape(1, 1))

    return jnp.concatenate([askm.reshape(-1), ss.reshape(-1),
                            scal.reshape(-1)])
